# baseline (speedup 1.0000x reference)
import jax
import jax.numpy as jnp
from jax.experimental import pallas as pl

TOPK = 100


def _decode_body(boxes_ref, scale_ref, out_ref):
    b = boxes_ref[...]
    cx, cy, w, h = b[..., 0], b[..., 1], b[..., 2], b[..., 3]
    xyxy = jnp.stack([cx - 0.5 * w, cy - 0.5 * h, cx + 0.5 * w, cy + 0.5 * h], axis=-1)
    out_ref[...] = xyxy * scale_ref[...]


def kernel(pred_logits, pred_boxes, target_sizes):
    B, N, C = pred_logits.shape
    prob = jax.nn.sigmoid(pred_logits)
    flat = prob.reshape(B, N * C)
    topk_values, topk_indexes = jax.lax.top_k(flat, TOPK)
    scores = topk_values
    topk_boxes = topk_indexes // C
    labels = topk_indexes % C
    gathered = jnp.take_along_axis(
        pred_boxes, jnp.repeat(topk_boxes[:, :, None], 4, axis=2), axis=1
    )
    img_h = target_sizes[:, 0].astype(jnp.float32)
    img_w = target_sizes[:, 1].astype(jnp.float32)
    scale_fct = jnp.stack([img_w, img_h, img_w, img_h], axis=1)
    scale = jnp.broadcast_to(scale_fct[:, None, :], (B, TOPK, 4))
    boxes = pl.pallas_call(
        _decode_body,
        out_shape=jax.ShapeDtypeStruct((B, TOPK, 4), jnp.float32),
    )(gathered, scale)
    return scores, labels, boxes


# speedup vs baseline: 12.0175x; 12.0175x over previous
import functools

import jax
import jax.numpy as jnp
from jax import lax
from jax.experimental import pallas as pl
from jax.experimental.pallas import tpu as pltpu
from jax.experimental.pallas import tpu_sc as plsc

B, N, C = 16, 20000, 91
TOPK = 100
CAP = 256
NBKT = 16384
NEG = -3.0e38


_RB = 2000


def _rowmax_body(x_ref, m_ref):
    x = x_ref[0]
    m_ref[0, 0] = jnp.max(x, axis=-1).reshape(8, _RB // 8)


def _rowmax(pred_logits):
    return pl.pallas_call(
        _rowmax_body,
        grid=(B, N // _RB),
        in_specs=[pl.BlockSpec((1, _RB, C), lambda b, i: (b, i, 0))],
        out_specs=pl.BlockSpec((1, 1, 8, _RB // 8), lambda b, i: (b, i, 0, 0)),
        out_shape=jax.ShapeDtypeStruct((B, N // _RB, 8, _RB // 8), jnp.float32),
    )(pred_logits).reshape(B, N)


def _order_bucket(v):
    u = plsc.bitcast(v, jnp.int32)
    neg = u < 0
    u2 = jnp.where(neg, ~u, u ^ jnp.int32(-2147483648))
    return lax.shift_right_logical(u2, 18)


def _iota16():
    return lax.iota(jnp.int32, 16)


def _find_edge(hist_ref, want):
    def body(t, carry):
        found, edge, ncnt, cum = carry
        j = NBKT // 16 - 1 - t
        vec = hist_ref[pl.ds(j * 16, 16)]
        rc = plsc.cumsum(lax.rev(vec.astype(jnp.float32), (0,)))
        rc = rc.astype(jnp.int32)
        tot = cum + rc
        cond = tot >= want
        lstar = jnp.min(jnp.where(cond, _iota16(), 99))
        hit = jnp.logical_and(jnp.logical_not(found), lstar < 16)
        cnt_here = jnp.sum(jnp.where(_iota16() == lstar, tot, 0))
        edge = jnp.where(hit, j * 16 + 15 - lstar, edge)
        ncnt = jnp.where(hit, cnt_here, ncnt)
        found = jnp.logical_or(found, hit)
        cum = cum + jnp.sum(vec)
        return found, edge, ncnt, cum
    _, edge, ncnt, _ = lax.fori_loop(
        0, NBKT // 16,
        body,
        (jnp.bool_(False), jnp.int32(0), jnp.int32(0), jnp.int32(0)),
    )
    return edge, ncnt


def _sc_body(m_hbm, logits_hbm, boxes_hbm, scale_hbm,
             val_out, lbl_out, box_out,
             m_v, hist, cand, rows_v, gidx, c2f, bidx, wboxes, lblbuf, c2v,
             bflat, scale_v, sem):
    cid = lax.axis_index("c")
    sid = lax.axis_index("s")
    wid = sid * 2 + cid
    b = wid

    @pl.when(wid < B)
    def _():
        pltpu.sync_copy(m_hbm.at[b], m_v)
        pltpu.sync_copy(scale_hbm.at[b], scale_v)

        def zero_body(i, _):
            hist[pl.ds(i * 16, 16)] = jnp.zeros((16,), jnp.int32)
            return 0
        lax.fori_loop(0, NBKT // 16, zero_body, 0)

        ones = jnp.ones((16,), jnp.int32)

        def hist_body(i, _):
            v = m_v[pl.ds(i * 16, 16)]
            plsc.addupdate_scatter(hist, [_order_bucket(v)], ones)
            return 0
        lax.fori_loop(0, N // 16, hist_body, 0)

        edge, _ = _find_edge(hist, TOPK)

        def cz_body(i, _):
            cand[pl.ds(i * 16, 16)] = jnp.zeros((16,), jnp.int32)
            return 0
        lax.fori_loop(0, CAP // 16, cz_body, 0)

        def compact_body(i, off):
            v = m_v[pl.ds(i * 16, 16)]
            keep = _order_bucket(v) >= edge
            ids = _iota16() + i * 16
            cnt = jnp.sum(keep.astype(jnp.int32))

            @pl.when(jnp.logical_and(cnt > 0, off <= CAP - 16))
            def _():
                plsc.store_compressed(cand.at[pl.ds(off, 16)], ids, mask=keep)
            return jnp.minimum(off + cnt, CAP)
        ncand = lax.fori_loop(0, N // 16, compact_body, jnp.int32(0))

        def gi_body(i, _):
            q = _iota16() + i * 16
            k = q // 7
            j = q - k * 7
            boxid = plsc.load_gather(cand, [k])
            gidx[pl.ds(i * 16, 16)] = lax.shift_right_logical(boxid * C, 4) + j
            return 0
        lax.fori_loop(0, CAP * 7 // 16, gi_body, 0)

        cps = []
        for mchunk in range(CAP * 7 // 128):
            cps.append(pltpu.async_copy(
                logits_hbm.at[b].at[gidx.at[pl.ds(mchunk * 128, 128)]],
                rows_v.at[pl.ds(mchunk * 128, 128)], sem))
        for cp in cps:
            cp.wait()

        lax.fori_loop(0, NBKT // 16, zero_body, 0)

        def hist2_body(i, _):
            p = _iota16() + i * 16
            row = p // C
            col = p - row * C
            boxid = plsc.load_gather(cand, [row])
            flat = boxid * C + col
            base = lax.shift_right_logical(boxid * C, 4)
            grow = row * 7 + lax.shift_right_logical(flat, 4) - base
            gcol = jnp.bitwise_and(flat, 15)
            v = plsc.load_gather(rows_v, [grow, gcol])
            bkt = jnp.where(row < ncand, _order_bucket(v), 0)
            plsc.addupdate_scatter(hist, [bkt], ones)
            return 0
        lax.fori_loop(0, CAP * C // 16, hist2_body, 0)

        edge2, _ = _find_edge(hist, TOPK)

        def c2z_body(i, _):
            c2v[pl.ds(i * 16, 16)] = jnp.full((16,), NEG, jnp.float32)
            c2f[pl.ds(i * 16, 16)] = jnp.zeros((16,), jnp.int32)
            return 0
        lax.fori_loop(0, CAP // 16, c2z_body, 0)

        def compact2_body(i, off):
            p = _iota16() + i * 16
            row = p // C
            col = p - row * C
            boxid = plsc.load_gather(cand, [row])
            fid = boxid * C + col
            base = lax.shift_right_logical(boxid * C, 4)
            grow = row * 7 + lax.shift_right_logical(fid, 4) - base
            gcol = jnp.bitwise_and(fid, 15)
            v = plsc.load_gather(rows_v, [grow, gcol])
            keep = jnp.logical_and(row < ncand, _order_bucket(v) >= edge2)
            cnt = jnp.sum(keep.astype(jnp.int32))

            @pl.when(jnp.logical_and(cnt > 0, off <= CAP - 16))
            def _():
                plsc.store_compressed(c2v.at[pl.ds(off, 16)], v, mask=keep)
                plsc.store_compressed(c2f.at[pl.ds(off, 16)], fid, mask=keep)
            return jnp.minimum(off + cnt, CAP)
        lax.fori_loop(0, CAP * C // 16, compact2_body, jnp.int32(0))

        def lb_body(k, _):
            fid = c2f[pl.ds(k * 16, 16)]
            bx = fid // C
            bidx[pl.ds(k * 16, 16)] = bx
            lblbuf[pl.ds(k * 16, 16)] = fid - bx * C
            return 0
        lax.fori_loop(0, CAP // 16, lb_body, 0)

        def bg_body(i, _):
            bx = bidx[pl.ds(i * 16, 16)]
            gidx[pl.ds(i * 16, 16)] = lax.shift_right_logical(bx, 2)
            return 0
        lax.fori_loop(0, CAP // 16, bg_body, 0)
        cp3 = pltpu.async_copy(
            boxes_hbm.at[b].at[gidx.at[pl.ds(0, 128)]],
            wboxes.at[pl.ds(0, 128)], sem)
        cp4 = pltpu.async_copy(
            boxes_hbm.at[b].at[gidx.at[pl.ds(128, 128)]],
            wboxes.at[pl.ds(128, 128)], sem)
        cp3.wait()
        cp4.wait()

        scale_vec = scale_v[pl.ds(0, 16)]

        def dec_body(k, _):
            l16 = _iota16()
            q = jnp.bitwise_and(l16, 3)
            row = k * 4 + lax.shift_right_logical(l16, 2)
            bx = plsc.load_gather(bidx, [row])
            boff = jnp.bitwise_and(bx, 3) * 4
            colA = boff + jnp.bitwise_and(q, 1)
            colB = colA + 2
            c1 = plsc.load_gather(wboxes, [row, colA])
            wh = plsc.load_gather(wboxes, [row, colB])
            sgn = jnp.where(q >= 2, jnp.float32(0.5), jnp.float32(-0.5))
            bflat[pl.ds(k * 16, 16)] = (c1 + sgn * wh) * scale_vec
            return 0
        lax.fori_loop(0, CAP * 4 // 16, dec_body, 0)

        pltpu.sync_copy(c2v, val_out.at[b])
        pltpu.sync_copy(lblbuf, lbl_out.at[b])
        pltpu.sync_copy(bflat, box_out.at[b])


def _sc_select(m, pred_logits, pred_boxes, scale):
    mesh = plsc.VectorSubcoreMesh(core_axis_name="c", subcore_axis_name="s")
    f = pl.kernel(
        _sc_body,
        out_type=(
            jax.ShapeDtypeStruct((B, CAP), jnp.float32),
            jax.ShapeDtypeStruct((B, CAP), jnp.int32),
            jax.ShapeDtypeStruct((B, CAP * 4), jnp.float32),
        ),
        mesh=mesh,
        compiler_params=pltpu.CompilerParams(needs_layout_passes=False, use_tc_tiling_on_sc=False),
        scratch_types=[
            pltpu.VMEM((N,), jnp.float32),
            pltpu.VMEM((NBKT,), jnp.int32),
            pltpu.VMEM((CAP,), jnp.int32),
            pltpu.VMEM((CAP * 7, 16), jnp.float32),
            pltpu.VMEM((CAP * 7,), jnp.int32),
            pltpu.VMEM((CAP,), jnp.int32),
            pltpu.VMEM((CAP,), jnp.int32),
            pltpu.VMEM((CAP, 16), jnp.float32),
            pltpu.VMEM((CAP,), jnp.int32),
            pltpu.VMEM((CAP,), jnp.float32),
            pltpu.VMEM((CAP * 4,), jnp.float32),
            pltpu.VMEM((16,), jnp.float32),
            pltpu.SemaphoreType.DMA,
        ],
    )
    return f(m, pred_logits.reshape(B, N * C // 16, 16),
             pred_boxes.reshape(B, N * 4 // 16, 16), scale)


def kernel(pred_logits, pred_boxes, target_sizes):
    m = _rowmax(pred_logits)

    img_h = target_sizes[:, 0].astype(jnp.float32)
    img_w = target_sizes[:, 1].astype(jnp.float32)
    scale = jnp.tile(jnp.stack([img_w, img_h, img_w, img_h], axis=1), (1, 4))

    vals, lbls, bflat = _sc_select(m, pred_logits, pred_boxes, scale)

    s = jax.nn.sigmoid(vals)
    scores, pos = jax.lax.top_k(s, TOPK)
    labels = jnp.take_along_axis(lbls, pos, axis=1)
    boxes = jnp.take_along_axis(
        bflat.reshape(B, CAP, 4),
        jnp.repeat(pos[:, :, None], 4, axis=2), axis=1)
    return scores, labels, boxes


# speedup vs baseline: 12.3382x; 1.0267x over previous
import functools

import jax
import jax.numpy as jnp
from jax import lax
from jax.experimental import pallas as pl
from jax.experimental.pallas import tpu as pltpu
from jax.experimental.pallas import tpu_sc as plsc

B, N, C = 16, 20000, 91
TOPK = 100
CAP = 256
NBKT = 16384
NEG = -3.0e38


_RB = 2000


def _rowmax_body(x_ref, m_ref):
    x = x_ref[0]
    m_ref[0, 0] = jnp.max(x, axis=-1).reshape(8, _RB // 8)


def _rowmax(pred_logits):
    return pl.pallas_call(
        _rowmax_body,
        grid=(B, N // _RB),
        in_specs=[pl.BlockSpec((1, _RB, C), lambda b, i: (b, i, 0))],
        out_specs=pl.BlockSpec((1, 1, 8, _RB // 8), lambda b, i: (b, i, 0, 0)),
        out_shape=jax.ShapeDtypeStruct((B, N // _RB, 8, _RB // 8), jnp.float32),
    )(pred_logits).reshape(B, N)


def _order_bucket(v):
    u = plsc.bitcast(v, jnp.int32)
    neg = u < 0
    u2 = jnp.where(neg, ~u, u ^ jnp.int32(-2147483648))
    return lax.shift_right_logical(u2, 18)


def _iota16():
    return lax.iota(jnp.int32, 16)


def _find_edge(hist_ref, want):
    def body(t, carry):
        found, edge, ncnt, cum = carry
        j = NBKT // 16 - 1 - t
        vec = hist_ref[pl.ds(j * 16, 16)]
        rc = plsc.cumsum(lax.rev(vec.astype(jnp.float32), (0,)))
        rc = rc.astype(jnp.int32)
        tot = cum + rc
        cond = tot >= want
        lstar = jnp.min(jnp.where(cond, _iota16(), 99))
        hit = jnp.logical_and(jnp.logical_not(found), lstar < 16)
        cnt_here = jnp.sum(jnp.where(_iota16() == lstar, tot, 0))
        edge = jnp.where(hit, j * 16 + 15 - lstar, edge)
        ncnt = jnp.where(hit, cnt_here, ncnt)
        found = jnp.logical_or(found, hit)
        cum = cum + jnp.sum(vec)
        return found, edge, ncnt, cum
    _, edge, ncnt, _ = lax.fori_loop(
        0, NBKT // 16,
        body,
        (jnp.bool_(False), jnp.int32(0), jnp.int32(0), jnp.int32(0)),
    )
    return edge, ncnt


def _sc_body(m_hbm, logits_hbm, boxes_hbm, scale_hbm,
             val_out, lbl_out, box_out,
             m_v, hist, cand, rows_v, gidx, c2f, bidx, wboxes, lblbuf, c2v,
             bflat, scale_v, sem):
    cid = lax.axis_index("c")
    sid = lax.axis_index("s")
    wid = sid * 2 + cid
    b = wid

    @pl.when(wid < B)
    def _():
        pltpu.sync_copy(m_hbm.at[b], m_v)
        pltpu.sync_copy(scale_hbm.at[b], scale_v)

        def zero_body(i, _):
            hist[pl.ds(i * 16, 16)] = jnp.zeros((16,), jnp.int32)
            return 0
        lax.fori_loop(0, NBKT // 16, zero_body, 0)

        ones = jnp.ones((16,), jnp.int32)

        def hist_body(i, _):
            v = m_v[pl.ds(i * 16, 16)]
            plsc.addupdate_scatter(hist, [_order_bucket(v)], ones)
            return 0
        lax.fori_loop(0, N // 16, hist_body, 0)

        edge, _ = _find_edge(hist, TOPK)

        def cz_body(i, _):
            cand[pl.ds(i * 16, 16)] = jnp.zeros((16,), jnp.int32)
            return 0
        lax.fori_loop(0, CAP // 16, cz_body, 0)

        def compact_body(i, off):
            v = m_v[pl.ds(i * 16, 16)]
            keep = _order_bucket(v) >= edge
            ids = _iota16() + i * 16
            cnt = jnp.sum(keep.astype(jnp.int32))

            @pl.when(jnp.logical_and(cnt > 0, off <= CAP - 16))
            def _():
                plsc.store_compressed(cand.at[pl.ds(off, 16)], ids, mask=keep)
            return jnp.minimum(off + cnt, CAP)
        ncand = lax.fori_loop(0, N // 16, compact_body, jnp.int32(0))

        def gi_body(i, _):
            q = _iota16() + i * 16
            k = q // 7
            j = q - k * 7
            boxid = plsc.load_gather(cand, [k])
            gidx[pl.ds(i * 16, 16)] = lax.shift_right_logical(boxid * C, 4) + j
            return 0
        lax.fori_loop(0, CAP * 7 // 16, gi_body, 0)

        cps = []
        for mchunk in range(CAP * 7 // 128):
            cps.append(pltpu.async_copy(
                logits_hbm.at[b].at[gidx.at[pl.ds(mchunk * 128, 128)]],
                rows_v.at[pl.ds(mchunk * 128, 128)], sem))
        for cp in cps:
            cp.wait()

        lax.fori_loop(0, NBKT // 16, zero_body, 0)

        def hist2_body(i, _):
            p = _iota16() + i * 16
            row = p // C
            col = p - row * C
            boxid = plsc.load_gather(cand, [row])
            flat = boxid * C + col
            base = lax.shift_right_logical(boxid * C, 4)
            grow = row * 7 + lax.shift_right_logical(flat, 4) - base
            gcol = jnp.bitwise_and(flat, 15)
            v = plsc.load_gather(rows_v, [grow, gcol])
            bkt = jnp.where(row < ncand, _order_bucket(v), 0)
            plsc.addupdate_scatter(hist, [bkt], ones)
            return 0
        n2 = (ncand * C + 15) // 16
        lax.fori_loop(0, n2, hist2_body, 0)

        edge2, _ = _find_edge(hist, TOPK)

        def c2z_body(i, _):
            c2v[pl.ds(i * 16, 16)] = jnp.full((16,), NEG, jnp.float32)
            c2f[pl.ds(i * 16, 16)] = jnp.zeros((16,), jnp.int32)
            return 0
        lax.fori_loop(0, CAP // 16, c2z_body, 0)

        def compact2_body(i, off):
            p = _iota16() + i * 16
            row = p // C
            col = p - row * C
            boxid = plsc.load_gather(cand, [row])
            fid = boxid * C + col
            base = lax.shift_right_logical(boxid * C, 4)
            grow = row * 7 + lax.shift_right_logical(fid, 4) - base
            gcol = jnp.bitwise_and(fid, 15)
            v = plsc.load_gather(rows_v, [grow, gcol])
            keep = jnp.logical_and(row < ncand, _order_bucket(v) >= edge2)
            cnt = jnp.sum(keep.astype(jnp.int32))

            @pl.when(jnp.logical_and(cnt > 0, off <= CAP - 16))
            def _():
                plsc.store_compressed(c2v.at[pl.ds(off, 16)], v, mask=keep)
                plsc.store_compressed(c2f.at[pl.ds(off, 16)], fid, mask=keep)
            return jnp.minimum(off + cnt, CAP)
        lax.fori_loop(0, n2, compact2_body, jnp.int32(0))

        def lb_body(k, _):
            fid = c2f[pl.ds(k * 16, 16)]
            bx = fid // C
            bidx[pl.ds(k * 16, 16)] = bx
            lblbuf[pl.ds(k * 16, 16)] = fid - bx * C
            return 0
        lax.fori_loop(0, CAP // 16, lb_body, 0)

        def bg_body(i, _):
            bx = bidx[pl.ds(i * 16, 16)]
            gidx[pl.ds(i * 16, 16)] = lax.shift_right_logical(bx, 2)
            return 0
        lax.fori_loop(0, CAP // 16, bg_body, 0)
        cp3 = pltpu.async_copy(
            boxes_hbm.at[b].at[gidx.at[pl.ds(0, 128)]],
            wboxes.at[pl.ds(0, 128)], sem)
        cp4 = pltpu.async_copy(
            boxes_hbm.at[b].at[gidx.at[pl.ds(128, 128)]],
            wboxes.at[pl.ds(128, 128)], sem)
        cp3.wait()
        cp4.wait()

        scale_vec = scale_v[pl.ds(0, 16)]

        def dec_body(k, _):
            l16 = _iota16()
            q = jnp.bitwise_and(l16, 3)
            row = k * 4 + lax.shift_right_logical(l16, 2)
            bx = plsc.load_gather(bidx, [row])
            boff = jnp.bitwise_and(bx, 3) * 4
            colA = boff + jnp.bitwise_and(q, 1)
            colB = colA + 2
            c1 = plsc.load_gather(wboxes, [row, colA])
            wh = plsc.load_gather(wboxes, [row, colB])
            sgn = jnp.where(q >= 2, jnp.float32(0.5), jnp.float32(-0.5))
            bflat[pl.ds(k * 16, 16)] = (c1 + sgn * wh) * scale_vec
            return 0
        lax.fori_loop(0, CAP * 4 // 16, dec_body, 0)

        pltpu.sync_copy(c2v, val_out.at[b])
        pltpu.sync_copy(lblbuf, lbl_out.at[b])
        pltpu.sync_copy(bflat, box_out.at[b])


def _sc_select(m, pred_logits, pred_boxes, scale):
    mesh = plsc.VectorSubcoreMesh(core_axis_name="c", subcore_axis_name="s")
    f = pl.kernel(
        _sc_body,
        out_type=(
            jax.ShapeDtypeStruct((B, CAP), jnp.float32),
            jax.ShapeDtypeStruct((B, CAP), jnp.int32),
            jax.ShapeDtypeStruct((B, CAP * 4), jnp.float32),
        ),
        mesh=mesh,
        compiler_params=pltpu.CompilerParams(needs_layout_passes=False, use_tc_tiling_on_sc=False),
        scratch_types=[
            pltpu.VMEM((N,), jnp.float32),
            pltpu.VMEM((NBKT,), jnp.int32),
            pltpu.VMEM((CAP,), jnp.int32),
            pltpu.VMEM((CAP * 7, 16), jnp.float32),
            pltpu.VMEM((CAP * 7,), jnp.int32),
            pltpu.VMEM((CAP,), jnp.int32),
            pltpu.VMEM((CAP,), jnp.int32),
            pltpu.VMEM((CAP, 16), jnp.float32),
            pltpu.VMEM((CAP,), jnp.int32),
            pltpu.VMEM((CAP,), jnp.float32),
            pltpu.VMEM((CAP * 4,), jnp.float32),
            pltpu.VMEM((16,), jnp.float32),
            pltpu.SemaphoreType.DMA,
        ],
    )
    return f(m, pred_logits.reshape(B, N * C // 16, 16),
             pred_boxes.reshape(B, N * 4 // 16, 16), scale)


def kernel(pred_logits, pred_boxes, target_sizes):
    m = _rowmax(pred_logits)

    img_h = target_sizes[:, 0].astype(jnp.float32)
    img_w = target_sizes[:, 1].astype(jnp.float32)
    scale = jnp.tile(jnp.stack([img_w, img_h, img_w, img_h], axis=1), (1, 4))

    vals, lbls, bflat = _sc_select(m, pred_logits, pred_boxes, scale)

    s = jax.nn.sigmoid(vals)
    scores, pos = jax.lax.top_k(s, TOPK)
    labels = jnp.take_along_axis(lbls, pos, axis=1)
    boxes = jnp.take_along_axis(
        bflat.reshape(B, CAP, 4),
        jnp.repeat(pos[:, :, None], 4, axis=2), axis=1)
    return scores, labels, boxes


# speedup vs baseline: 12.6352x; 1.0241x over previous
import functools

import jax
import jax.numpy as jnp
from jax import lax
from jax.experimental import pallas as pl
from jax.experimental.pallas import tpu as pltpu
from jax.experimental.pallas import tpu_sc as plsc

B, N, C = 16, 20000, 91
TOPK = 100
CAP = 256
NBKT = 16384
NEG = -3.0e38


_RB = 2000


def _rowmax_body(x_ref, m_ref):
    x = x_ref[0]
    m_ref[0, 0] = jnp.max(x, axis=-1).reshape(8, _RB // 8)


def _rowmax(pred_logits):
    return pl.pallas_call(
        _rowmax_body,
        grid=(B, N // _RB),
        in_specs=[pl.BlockSpec((1, _RB, C), lambda b, i: (b, i, 0))],
        out_specs=pl.BlockSpec((1, 1, 8, _RB // 8), lambda b, i: (b, i, 0, 0)),
        out_shape=jax.ShapeDtypeStruct((B, N // _RB, 8, _RB // 8), jnp.float32),
    )(pred_logits).reshape(B, N)


def _order_bucket(v):
    u = plsc.bitcast(v, jnp.int32)
    neg = u < 0
    u2 = jnp.where(neg, ~u, u ^ jnp.int32(-2147483648))
    return lax.shift_right_logical(u2, 18)


def _iota16():
    return lax.iota(jnp.int32, 16)


def _find_edge(hist_ref, want):
    def body(t, carry):
        found, edge, ncnt, cum = carry
        j = NBKT // 16 - 1 - t
        vec = hist_ref[pl.ds(j * 16, 16)]
        rc = plsc.cumsum(lax.rev(vec.astype(jnp.float32), (0,)))
        rc = rc.astype(jnp.int32)
        tot = cum + rc
        cond = tot >= want
        lstar = jnp.min(jnp.where(cond, _iota16(), 99))
        hit = jnp.logical_and(jnp.logical_not(found), lstar < 16)
        cnt_here = jnp.sum(jnp.where(_iota16() == lstar, tot, 0))
        edge = jnp.where(hit, j * 16 + 15 - lstar, edge)
        ncnt = jnp.where(hit, cnt_here, ncnt)
        found = jnp.logical_or(found, hit)
        cum = cum + jnp.sum(vec)
        return found, edge, ncnt, cum
    _, edge, ncnt, _ = lax.fori_loop(
        0, NBKT // 16,
        body,
        (jnp.bool_(False), jnp.int32(0), jnp.int32(0), jnp.int32(0)),
    )
    return edge, ncnt


def _sc_body(m_hbm, logits_hbm, boxes_hbm, scale_hbm,
             val_out, lbl_out, box_out,
             m_v, hist, cand, rows_v, gidx, c2f, bidx, wboxes, lblbuf, c2v,
             bflat, scale_v, sem):
    cid = lax.axis_index("c")
    sid = lax.axis_index("s")
    wid = sid * 2 + cid
    b = wid

    @pl.when(wid < B)
    def _():
        pltpu.sync_copy(m_hbm.at[b], m_v)
        pltpu.sync_copy(scale_hbm.at[b], scale_v)

        def zero_body(i, _):
            for u in range(4):
                hist[pl.ds(i * 64 + u * 16, 16)] = jnp.zeros((16,), jnp.int32)
            return 0
        lax.fori_loop(0, NBKT // 64, zero_body, 0)

        ones = jnp.ones((16,), jnp.int32)

        def hist_body(i, _):
            for u in range(4):
                v = m_v[pl.ds(i * 64 + u * 16, 16)]
                plsc.addupdate_scatter(hist, [_order_bucket(v)], ones)
            return 0
        lax.fori_loop(0, N // 64, hist_body, 0)
        for u in range(2):
            v = m_v[pl.ds((N // 64) * 64 + u * 16, 16)]
            plsc.addupdate_scatter(hist, [_order_bucket(v)], ones)

        edge, _ = _find_edge(hist, TOPK)

        def cz_body(i, _):
            cand[pl.ds(i * 16, 16)] = jnp.zeros((16,), jnp.int32)
            return 0
        lax.fori_loop(0, CAP // 16, cz_body, 0)

        def compact_body(i, off):
            v = m_v[pl.ds(i * 16, 16)]
            keep = _order_bucket(v) >= edge
            ids = _iota16() + i * 16
            cnt = jnp.sum(keep.astype(jnp.int32))

            @pl.when(jnp.logical_and(cnt > 0, off <= CAP - 16))
            def _():
                plsc.store_compressed(cand.at[pl.ds(off, 16)], ids, mask=keep)
            return jnp.minimum(off + cnt, CAP)
        ncand = lax.fori_loop(0, N // 16, compact_body, jnp.int32(0))

        def gi_body(i, _):
            for u in range(4):
                q = _iota16() + i * 64 + u * 16
                k = q // 7
                j = q - k * 7
                boxid = plsc.load_gather(cand, [k])
                gidx[pl.ds(i * 64 + u * 16, 16)] = (
                    lax.shift_right_logical(boxid * C, 4) + j)
            return 0
        lax.fori_loop(0, CAP * 7 // 64, gi_body, 0)

        cps = []
        for mchunk in range(CAP * 7 // 128):
            cps.append(pltpu.async_copy(
                logits_hbm.at[b].at[gidx.at[pl.ds(mchunk * 128, 128)]],
                rows_v.at[pl.ds(mchunk * 128, 128)], sem))
        for cp in cps:
            cp.wait()

        lax.fori_loop(0, NBKT // 64, zero_body, 0)

        def hist2_body(i, _):
            for u in range(4):
                p = _iota16() + i * 64 + u * 16
                row = p // C
                col = p - row * C
                boxid = plsc.load_gather(cand, [row])
                flat = boxid * C + col
                base = lax.shift_right_logical(boxid * C, 4)
                grow = row * 7 + lax.shift_right_logical(flat, 4) - base
                gcol = jnp.bitwise_and(flat, 15)
                v = plsc.load_gather(rows_v, [grow, gcol])
                bkt = jnp.where(row < ncand, _order_bucket(v), 0)
                plsc.addupdate_scatter(hist, [bkt], ones)
            return 0
        n2u = (ncand * C + 63) // 64
        lax.fori_loop(0, n2u, hist2_body, 0)
        n2 = (ncand * C + 15) // 16

        edge2, _ = _find_edge(hist, TOPK)

        def c2z_body(i, _):
            c2v[pl.ds(i * 16, 16)] = jnp.full((16,), NEG, jnp.float32)
            c2f[pl.ds(i * 16, 16)] = jnp.zeros((16,), jnp.int32)
            return 0
        lax.fori_loop(0, CAP // 16, c2z_body, 0)

        def compact2_body(i, off):
            p = _iota16() + i * 16
            row = p // C
            col = p - row * C
            boxid = plsc.load_gather(cand, [row])
            fid = boxid * C + col
            base = lax.shift_right_logical(boxid * C, 4)
            grow = row * 7 + lax.shift_right_logical(fid, 4) - base
            gcol = jnp.bitwise_and(fid, 15)
            v = plsc.load_gather(rows_v, [grow, gcol])
            keep = jnp.logical_and(row < ncand, _order_bucket(v) >= edge2)
            cnt = jnp.sum(keep.astype(jnp.int32))

            @pl.when(jnp.logical_and(cnt > 0, off <= CAP - 16))
            def _():
                plsc.store_compressed(c2v.at[pl.ds(off, 16)], v, mask=keep)
                plsc.store_compressed(c2f.at[pl.ds(off, 16)], fid, mask=keep)
            return jnp.minimum(off + cnt, CAP)
        lax.fori_loop(0, n2, compact2_body, jnp.int32(0))

        def lb_body(k, _):
            fid = c2f[pl.ds(k * 16, 16)]
            bx = fid // C
            bidx[pl.ds(k * 16, 16)] = bx
            lblbuf[pl.ds(k * 16, 16)] = fid - bx * C
            return 0
        lax.fori_loop(0, CAP // 16, lb_body, 0)

        def bg_body(i, _):
            bx = bidx[pl.ds(i * 16, 16)]
            gidx[pl.ds(i * 16, 16)] = lax.shift_right_logical(bx, 2)
            return 0
        lax.fori_loop(0, CAP // 16, bg_body, 0)
        cp3 = pltpu.async_copy(
            boxes_hbm.at[b].at[gidx.at[pl.ds(0, 128)]],
            wboxes.at[pl.ds(0, 128)], sem)
        cp4 = pltpu.async_copy(
            boxes_hbm.at[b].at[gidx.at[pl.ds(128, 128)]],
            wboxes.at[pl.ds(128, 128)], sem)
        cp3.wait()
        cp4.wait()

        scale_vec = scale_v[pl.ds(0, 16)]

        def dec_body(k, _):
            l16 = _iota16()
            q = jnp.bitwise_and(l16, 3)
            row = k * 4 + lax.shift_right_logical(l16, 2)
            bx = plsc.load_gather(bidx, [row])
            boff = jnp.bitwise_and(bx, 3) * 4
            colA = boff + jnp.bitwise_and(q, 1)
            colB = colA + 2
            c1 = plsc.load_gather(wboxes, [row, colA])
            wh = plsc.load_gather(wboxes, [row, colB])
            sgn = jnp.where(q >= 2, jnp.float32(0.5), jnp.float32(-0.5))
            bflat[pl.ds(k * 16, 16)] = (c1 + sgn * wh) * scale_vec
            return 0
        lax.fori_loop(0, CAP * 4 // 16, dec_body, 0)

        pltpu.sync_copy(c2v, val_out.at[b])
        pltpu.sync_copy(lblbuf, lbl_out.at[b])
        pltpu.sync_copy(bflat, box_out.at[b])


def _sc_select(m, pred_logits, pred_boxes, scale):
    mesh = plsc.VectorSubcoreMesh(core_axis_name="c", subcore_axis_name="s")
    f = pl.kernel(
        _sc_body,
        out_type=(
            jax.ShapeDtypeStruct((B, CAP), jnp.float32),
            jax.ShapeDtypeStruct((B, CAP), jnp.int32),
            jax.ShapeDtypeStruct((B, CAP * 4), jnp.float32),
        ),
        mesh=mesh,
        compiler_params=pltpu.CompilerParams(needs_layout_passes=False, use_tc_tiling_on_sc=False),
        scratch_types=[
            pltpu.VMEM((N,), jnp.float32),
            pltpu.VMEM((NBKT,), jnp.int32),
            pltpu.VMEM((CAP,), jnp.int32),
            pltpu.VMEM((CAP * 7, 16), jnp.float32),
            pltpu.VMEM((CAP * 7,), jnp.int32),
            pltpu.VMEM((CAP,), jnp.int32),
            pltpu.VMEM((CAP,), jnp.int32),
            pltpu.VMEM((CAP, 16), jnp.float32),
            pltpu.VMEM((CAP,), jnp.int32),
            pltpu.VMEM((CAP,), jnp.float32),
            pltpu.VMEM((CAP * 4,), jnp.float32),
            pltpu.VMEM((16,), jnp.float32),
            pltpu.SemaphoreType.DMA,
        ],
    )
    return f(m, pred_logits.reshape(B, N * C // 16, 16),
             pred_boxes.reshape(B, N * 4 // 16, 16), scale)


def kernel(pred_logits, pred_boxes, target_sizes):
    m = _rowmax(pred_logits)

    img_h = target_sizes[:, 0].astype(jnp.float32)
    img_w = target_sizes[:, 1].astype(jnp.float32)
    scale = jnp.tile(jnp.stack([img_w, img_h, img_w, img_h], axis=1), (1, 4))

    vals, lbls, bflat = _sc_select(m, pred_logits, pred_boxes, scale)

    s = jax.nn.sigmoid(vals)
    scores, pos = jax.lax.top_k(s, TOPK)
    labels = jnp.take_along_axis(lbls, pos, axis=1)
    boxes = jnp.take_along_axis(
        bflat.reshape(B, CAP, 4),
        jnp.repeat(pos[:, :, None], 4, axis=2), axis=1)
    return scores, labels, boxes
